# Initial kernel scaffold; baseline (speedup 1.0000x reference)
import jax
import jax.numpy as jnp
from jax.experimental import pallas as pl


def kernel(x_int, x_lane, x_sens, x_inj, Wp_int, bp_int, Wp_lane, bp_lane, Wp_sens, bp_sens, Wp_inj, bp_inj, W_self, b_self, W_spatial, W_flow, W_incident, spatial_e, flow_lane_e, flow_sens_e, incident_e):
    raise NotImplementedError("write your pallas kernel here")


# speedup vs baseline: 2.5030x; 2.5030x over previous
import functools

import jax
import jax.numpy as jnp
from jax import lax
from jax.experimental import pallas as pl
from jax.experimental.pallas import tpu as pltpu
from jax.experimental.pallas import tpu_sc as plsc

_N = 50000
_H = 128
_L = 3
_NSC = 2
_NSUB = 16
_KEDGE = 128
_DUMMY = 48
_EALIGN = _NSC * _NSUB * _KEDGE
_NP = _N + _DUMMY


def _pad_edges(e):
    return ((e + _EALIGN - 1) // _EALIGN) * _EALIGN


def _make_seg_sum(n_src, n_out, e_pad, cw, nch):
    ch_per_sc = (nch + 1) // 2
    ept = e_pad // _NSUB
    nblk = ept // _KEDGE
    acc_rows = n_out + _DUMMY
    zrows = acc_rows // _NSUB

    def body(tab_ref, srcq_ref, dst_ref, zeros_ref, out_ref,
             srcv, dstv, rows, acc, sem):
        c = lax.axis_index("c")
        s = lax.axis_index("s")

        def chunk_body(q):
            pltpu.sync_copy(zeros_ref.at[pl.ds(s * zrows, zrows)],
                            acc.at[pl.ds(s * zrows, zrows)])
            plsc.subcore_barrier()

            def eb(i, carry):
                base = s * ept + i * _KEDGE
                pltpu.sync_copy(srcq_ref.at[pl.ds(q * e_pad + base, _KEDGE)],
                                srcv)
                pltpu.sync_copy(dst_ref.at[pl.ds(base, _KEDGE)], dstv)
                pltpu.async_copy(tab_ref.at[srcv], rows, sem).wait()
                pltpu.sync_copy(rows, acc.at[dstv], add=True)
                return carry

            lax.fori_loop(0, nblk, eb, 0)
            plsc.subcore_barrier()
            pltpu.sync_copy(
                acc.at[pl.ds(s * zrows, zrows)],
                out_ref.at[pl.ds(q * acc_rows + s * zrows, zrows)])

        for j in range(ch_per_sc):
            q = c * ch_per_sc + j
            if ch_per_sc * 2 == nch:
                chunk_body(q)
            else:
                pl.when(q < nch)(lambda: chunk_body(q))

    return pl.kernel(
        body,
        out_type=jax.ShapeDtypeStruct((nch * acc_rows, cw), jnp.float32),
        mesh=plsc.VectorSubcoreMesh(core_axis_name="c", subcore_axis_name="s"),
        scratch_types=[
            pltpu.VMEM((_KEDGE,), jnp.int32),
            pltpu.VMEM((_KEDGE,), jnp.int32),
            pltpu.VMEM((_KEDGE, cw), jnp.float32),
            pltpu.VMEM_SHARED((acc_rows, cw), jnp.float32),
            pltpu.SemaphoreType.DMA,
        ],
        compiler_params=pltpu.CompilerParams(use_tc_tiling_on_sc=False),
    )


def _make_counts(n_out, e_pad):
    cw = 16
    ept = e_pad // (_NSC * _NSUB)
    nblk = ept // _KEDGE
    acc_rows = n_out + _DUMMY
    zrows = acc_rows // _NSUB

    def body(dst_ref, zeros_ref, out_ref, dstv, ones, acc, sem):
        del sem
        c = lax.axis_index("c")
        s = lax.axis_index("s")
        for i in range(_KEDGE):
            ones[i, :] = jnp.full((cw,), 1.0, jnp.float32)
        pltpu.sync_copy(zeros_ref.at[pl.ds(s * zrows, zrows)],
                        acc.at[pl.ds(s * zrows, zrows)])
        plsc.subcore_barrier()

        def eb(i, carry):
            base = (c * _NSUB + s) * ept + i * _KEDGE
            pltpu.sync_copy(dst_ref.at[pl.ds(base, _KEDGE)], dstv)
            pltpu.sync_copy(ones, acc.at[dstv], add=True)
            return carry

        lax.fori_loop(0, nblk, eb, 0)
        plsc.subcore_barrier()
        pltpu.sync_copy(acc.at[pl.ds(s * zrows, zrows)],
                        out_ref.at[pl.ds(c * acc_rows + s * zrows, zrows)])

    return pl.kernel(
        body,
        out_type=jax.ShapeDtypeStruct((_NSC * acc_rows, cw), jnp.float32),
        mesh=plsc.VectorSubcoreMesh(core_axis_name="c", subcore_axis_name="s"),
        scratch_types=[
            pltpu.VMEM((_KEDGE,), jnp.int32),
            pltpu.VMEM((_KEDGE, cw), jnp.float32),
            pltpu.VMEM_SHARED((acc_rows, cw), jnp.float32),
            pltpu.SemaphoreType.DMA,
        ],
        compiler_params=pltpu.CompilerParams(use_tc_tiling_on_sc=False),
    )


_BM = 1000


def _proj_int_body(x_ref, w_ref, b_ref, out_ref, outq_ref):
    y = jnp.dot(x_ref[...], w_ref[...],
                preferred_element_type=jnp.float32) + b_ref[...]
    out_ref[...] = y
    for q in range(4):
        outq_ref[q] = y[:, 32 * q:32 * (q + 1)]


def _proj_int(x, w, b):
    nb = x.shape[0] // _BM
    return pl.pallas_call(
        _proj_int_body,
        grid=(nb,),
        in_specs=[
            pl.BlockSpec((_BM, 128), lambda i: (i, 0)),
            pl.BlockSpec((128, 128), lambda i: (0, 0)),
            pl.BlockSpec((1, 128), lambda i: (0, 0)),
        ],
        out_specs=[
            pl.BlockSpec((_BM, 128), lambda i: (i, 0)),
            pl.BlockSpec((4, _BM, 32), lambda i: (0, i, 0)),
        ],
        out_shape=[
            jax.ShapeDtypeStruct((x.shape[0], 128), jnp.float32),
            jax.ShapeDtypeStruct((4, x.shape[0], 32), jnp.float32),
        ],
    )(x, w, b)


def _csum(cnt_blk):
    return cnt_blk[0, :, 0:1] + cnt_blk[1, :, 0:1]


def _const_build_body(al_ref, as_ref, ai_ref, cl_ref, cs_ref, ci_ref,
                      wls_ref, winj_ref, bl_ref, bs_ref, bi_ref, wfwi_ref,
                      out_ref):
    cl = _csum(cl_ref[...])
    cs = _csum(cs_ref[...])
    ci = _csum(ci_ref[...])
    rl = 1.0 / jnp.maximum(cl, 1.0)
    rs = 1.0 / jnp.maximum(cs, 1.0)
    ri = 1.0 / jnp.maximum(ci, 1.0)
    albk = al_ref[...]
    asbk = as_ref[...]
    ml = jnp.concatenate([albk[q] for q in range(4)], axis=1) * rl
    ms = jnp.concatenate([asbk[q] for q in range(2)], axis=1) * rs
    mi = ai_ref[0] * ri
    u = jnp.dot(jnp.concatenate([ml, ms], axis=1), wls_ref[...],
                preferred_element_type=jnp.float32)
    u = u + jnp.where(cl > 0, 1.0, 0.0) * bl_ref[...]
    u = u + jnp.where(cs > 0, 1.0, 0.0) * bs_ref[...]
    v = jnp.dot(mi, winj_ref[...], preferred_element_type=jnp.float32)
    v = v + jnp.where(ci > 0, 1.0, 0.0) * bi_ref[...]
    out_ref[...] = jnp.dot(jnp.concatenate([u, v], axis=1), wfwi_ref[...],
                           preferred_element_type=jnp.float32)


def _const_build(agg_lane, agg_sens, agg_inj, cnt_fl, cnt_fs, cnt_inc,
                 w_ls, wp_inj, b_lane, b_sens, b_inj, wfwi):
    nb = _N // _BM
    return pl.pallas_call(
        _const_build_body,
        grid=(nb,),
        in_specs=[
            pl.BlockSpec((4, _BM, 16), lambda i: (0, i, 0)),
            pl.BlockSpec((2, _BM, 16), lambda i: (0, i, 0)),
            pl.BlockSpec((1, _BM, 16), lambda i: (0, i, 0)),
            pl.BlockSpec((2, _BM, 16), lambda i: (0, i, 0)),
            pl.BlockSpec((2, _BM, 16), lambda i: (0, i, 0)),
            pl.BlockSpec((2, _BM, 16), lambda i: (0, i, 0)),
            pl.BlockSpec((96, 128), lambda i: (0, 0)),
            pl.BlockSpec((16, 128), lambda i: (0, 0)),
            pl.BlockSpec((1, 128), lambda i: (0, 0)),
            pl.BlockSpec((1, 128), lambda i: (0, 0)),
            pl.BlockSpec((1, 128), lambda i: (0, 0)),
            pl.BlockSpec((256, 384), lambda i: (0, 0)),
        ],
        out_specs=pl.BlockSpec((_BM, 384), lambda i: (i, 0)),
        out_shape=jax.ShapeDtypeStruct((_N, 384), jnp.float32),
    )(agg_lane, agg_sens, agg_inj, cnt_fl, cnt_fs, cnt_inc,
      w_ls, wp_inj, b_lane, b_sens, b_inj, wfwi)


def _combine_body(h_ref, sp_ref, csp_ref, const_ref, ws_ref, wsp_ref, b_ref,
                  out_ref, outq_ref):
    csp = _csum(csp_ref[...])
    rsp = 1.0 / jnp.maximum(csp, 1.0)
    spbk = sp_ref[...]
    msp = jnp.concatenate([spbk[q] for q in range(4)], axis=1) * rsp
    y = jnp.dot(h_ref[...], ws_ref[...], preferred_element_type=jnp.float32)
    y = y + jnp.dot(msp, wsp_ref[...], preferred_element_type=jnp.float32)
    y = y + b_ref[...] + const_ref[...]
    y = jnp.where(y > 0, y, jnp.exp(jnp.minimum(y, 0.0)) - 1.0)
    out_ref[...] = y
    for q in range(4):
        outq_ref[q] = y[:, 32 * q:32 * (q + 1)]


def _combine(h, agg_sp, cnt_sp, const_all, w_self_l, w_sp_l, b_l, lidx):
    nb = _N // _BM
    return pl.pallas_call(
        _combine_body,
        grid=(nb,),
        in_specs=[
            pl.BlockSpec((_BM, 128), lambda i: (i, 0)),
            pl.BlockSpec((4, _BM, 32), lambda i: (0, i, 0)),
            pl.BlockSpec((2, _BM, 16), lambda i: (0, i, 0)),
            pl.BlockSpec((_BM, 128), lambda i, _l=lidx: (i, _l)),
            pl.BlockSpec((128, 128), lambda i: (0, 0)),
            pl.BlockSpec((128, 128), lambda i: (0, 0)),
            pl.BlockSpec((1, 128), lambda i: (0, 0)),
        ],
        out_specs=[
            pl.BlockSpec((_BM, 128), lambda i: (i, 0)),
            pl.BlockSpec((4, _BM, 32), lambda i: (0, i, 0)),
        ],
        out_shape=[
            jax.ShapeDtypeStruct((_N, 128), jnp.float32),
            jax.ShapeDtypeStruct((4, _N, 32), jnp.float32),
        ],
    )(h, agg_sp, cnt_sp, const_all, w_self_l, w_sp_l, b_l)


def _prep_edges(edges, n_src, nch):
    src, dst = edges[0], edges[1]
    e = src.shape[0]
    e_pad = _pad_edges(e)
    pad = e_pad - e
    src_p = jnp.concatenate([src, jnp.zeros((pad,), jnp.int32)])
    dst_p = jnp.concatenate(
        [dst, _N + (jnp.arange(pad, dtype=jnp.int32) % _DUMMY)])
    srcq = (src_p[None, :]
            + (jnp.arange(nch, dtype=jnp.int32) * n_src)[:, None]).reshape(-1)
    return srcq, dst_p, e_pad


def _chunk16(x, nch):
    n = x.shape[0]
    return jnp.transpose(x.reshape(n, nch, 16), (1, 0, 2)).reshape(nch * n, 16)


def kernel(x_int, x_lane, x_sens, x_inj, Wp_int, bp_int, Wp_lane, bp_lane,
           Wp_sens, bp_sens, Wp_inj, bp_inj, W_self, b_self, W_spatial,
           W_flow, W_incident, spatial_e, flow_lane_e, flow_sens_e,
           incident_e):
    f32 = jnp.float32
    zeros16 = jnp.zeros((_N + _DUMMY, 16), f32)
    zeros32 = jnp.zeros((_N + _DUMMY, 32), f32)

    sp_srcq, sp_dst, sp_ep = _prep_edges(spatial_e, _N, 4)
    fl_srcq, fl_dst, fl_ep = _prep_edges(flow_lane_e, 50000, 4)
    fs_srcq, fs_dst, fs_ep = _prep_edges(flow_sens_e, 20000, 2)
    inc_srcq, inc_dst, inc_ep = _prep_edges(incident_e, 10000, 1)

    cnt_sp = _make_counts(_N, sp_ep)(sp_dst, zeros16)
    cnt_fl = _make_counts(_N, fl_ep)(fl_dst, zeros16)
    cnt_fs = _make_counts(_N, fs_ep)(fs_dst, zeros16)
    cnt_inc = _make_counts(_N, inc_ep)(inc_dst, zeros16)

    agg_lane = _make_seg_sum(50000, _N, fl_ep, 16, 4)(
        _chunk16(x_lane, 4), fl_srcq, fl_dst, zeros16)
    agg_sens = _make_seg_sum(20000, _N, fs_ep, 16, 2)(
        _chunk16(x_sens, 2), fs_srcq, fs_dst, zeros16)
    agg_inj = _make_seg_sum(10000, _N, inc_ep, 16, 1)(
        _chunk16(x_inj, 1), inc_srcq, inc_dst, zeros16)

    h, hq = _proj_int(x_int, Wp_int, bp_int.reshape(1, 128))

    w_ls = jnp.concatenate([Wp_lane, Wp_sens], axis=0)
    wf3 = jnp.transpose(W_flow, (1, 0, 2)).reshape(128, 384)
    wi3 = jnp.transpose(W_incident, (1, 0, 2)).reshape(128, 384)
    wfwi = jnp.concatenate([wf3, wi3], axis=0)
    const_all = _const_build(
        agg_lane.reshape(4, _NP, 16), agg_sens.reshape(2, _NP, 16),
        agg_inj.reshape(1, _NP, 16),
        cnt_fl.reshape(2, _NP, 16), cnt_fs.reshape(2, _NP, 16),
        cnt_inc.reshape(2, _NP, 16),
        w_ls, Wp_inj, bp_lane.reshape(1, 128), bp_sens.reshape(1, 128),
        bp_inj.reshape(1, 128), wfwi)

    cnt_sp_r = cnt_sp.reshape(2, _NP, 16)
    seg_sp = _make_seg_sum(_N, _N, sp_ep, 32, 4)
    for l in range(_L):
        agg_sp = seg_sp(hq.reshape(4 * _N, 32), sp_srcq, sp_dst, zeros32)
        h, hq = _combine(h, agg_sp.reshape(4, _NP, 32), cnt_sp_r, const_all,
                         W_self[l], W_spatial[l], b_self[l].reshape(1, 128),
                         l)
    return h


# speedup vs baseline: 3.6768x; 1.4690x over previous
import functools

import jax
import jax.numpy as jnp
from jax import lax
from jax.experimental import pallas as pl
from jax.experimental.pallas import tpu as pltpu
from jax.experimental.pallas import tpu_sc as plsc

_N = 50000
_H = 128
_L = 3
_NSC = 2
_NSUB = 16
_KEDGE = 128
_DUMMY = 48
_EALIGN = _NSC * _NSUB * _KEDGE
_NP = _N + _DUMMY


def _pad_edges(e):
    return ((e + _EALIGN - 1) // _EALIGN) * _EALIGN


def _make_seg_sum(n_src, n_out, e_pad, cw, nch):
    ch_per_sc = (nch + 1) // 2
    ept = e_pad // _NSUB
    nblk = ept // _KEDGE
    acc_rows = n_out + _DUMMY
    zrows = acc_rows // _NSUB

    def body(tab_ref, srcq_ref, dst_ref, zeros_ref, out_ref,
             srcv0, srcv1, srcv2, dstv0, dstv1, dstv2,
             rows0, rows1, rows2, acc,
             si0, si1, si2, sg0, sg1, sg2, ss0, ss1, ss2):
        srcv = (srcv0, srcv1, srcv2)
        dstv = (dstv0, dstv1, dstv2)
        rows = (rows0, rows1, rows2)
        si = (si0, si1, si2)
        sg = (sg0, sg1, sg2)
        ss = (ss0, ss1, ss2)
        c = lax.axis_index("c")
        s = lax.axis_index("s")

        def chunk_body(q):
            def src_sl(i):
                return srcq_ref.at[pl.ds(q * e_pad + s * ept + i * _KEDGE,
                                         _KEDGE)]

            def dst_sl(i):
                return dst_ref.at[pl.ds(s * ept + i * _KEDGE, _KEDGE)]

            def start_idx(i, b):
                pltpu.async_copy(src_sl(i), srcv[b], si[b])
                pltpu.async_copy(dst_sl(i), dstv[b], si[b])

            def wait_idx(i, b):
                pltpu.make_async_copy(src_sl(i), srcv[b], si[b]).wait()
                pltpu.make_async_copy(dst_sl(i), dstv[b], si[b]).wait()

            def start_gather(b):
                pltpu.async_copy(tab_ref.at[srcv[b]], rows[b], sg[b])

            def wait_gather(b):
                pltpu.make_async_copy(tab_ref.at[srcv[b]], rows[b],
                                      sg[b]).wait()

            def start_scatter(b):
                pltpu.async_copy(rows[b], acc.at[dstv[b]], ss[b], add=True)

            def wait_scatter(b):
                pltpu.make_async_copy(rows[b], acc.at[dstv[b]], ss[b]).wait()

            pltpu.sync_copy(zeros_ref.at[pl.ds(s * zrows, zrows)],
                            acc.at[pl.ds(s * zrows, zrows)])
            plsc.subcore_barrier()

            def step(i, b, static):
                wait_gather(b)
                start_scatter(b)
                b1, b2 = (b + 1) % 3, (b + 2) % 3

                def do_ws():
                    wait_scatter(b2)

                def do_idx():
                    start_idx(i + 2, b2)

                def do_g():
                    wait_idx(i + 1, b1)
                    start_gather(b1)

                if static:
                    if i > 0:
                        do_ws()
                    if i + 2 < nblk:
                        do_idx()
                    if i + 1 < nblk:
                        do_g()
                else:
                    pl.when(i > 0)(do_ws)
                    pl.when(i + 2 < nblk)(do_idx)
                    pl.when(i + 1 < nblk)(do_g)

            start_idx(0, 0)
            if nblk > 1:
                start_idx(1, 1)
            wait_idx(0, 0)
            start_gather(0)
            nsup, rem = nblk // 3, nblk % 3

            def sup_body(sup, carry):
                for j in range(3):
                    step(sup * 3 + j, j, False)
                return carry

            lax.fori_loop(0, nsup, sup_body, 0)
            for j in range(rem):
                step(nsup * 3 + j, j, True)
            wait_scatter((nblk - 1) % 3)
            plsc.subcore_barrier()
            pltpu.sync_copy(
                acc.at[pl.ds(s * zrows, zrows)],
                out_ref.at[pl.ds(q * acc_rows + s * zrows, zrows)])

        for j in range(ch_per_sc):
            q = c * ch_per_sc + j
            if ch_per_sc * 2 == nch:
                chunk_body(q)
            else:
                pl.when(q < nch)(lambda: chunk_body(q))

    return pl.kernel(
        body,
        out_type=jax.ShapeDtypeStruct((nch * acc_rows, cw), jnp.float32),
        mesh=plsc.VectorSubcoreMesh(core_axis_name="c", subcore_axis_name="s"),
        scratch_types=(
            [pltpu.VMEM((_KEDGE,), jnp.int32) for _ in range(6)]
            + [pltpu.VMEM((_KEDGE, cw), jnp.float32) for _ in range(3)]
            + [pltpu.VMEM_SHARED((acc_rows, cw), jnp.float32)]
            + [pltpu.SemaphoreType.DMA for _ in range(9)]
        ),
        compiler_params=pltpu.CompilerParams(use_tc_tiling_on_sc=False),
    )


def _make_counts(n_out, e_pad):
    cw = 16
    ept = e_pad // (_NSC * _NSUB)
    nblk = ept // _KEDGE
    acc_rows = n_out + _DUMMY
    zrows = acc_rows // _NSUB

    def body(dst_ref, zeros_ref, out_ref, dstv0, dstv1, ones, acc,
             si0, si1, ss0, ss1):
        dstv = (dstv0, dstv1)
        si = (si0, si1)
        ss = (ss0, ss1)
        c = lax.axis_index("c")
        s = lax.axis_index("s")
        for i in range(_KEDGE):
            ones[i, :] = jnp.full((cw,), 1.0, jnp.float32)
        pltpu.sync_copy(zeros_ref.at[pl.ds(s * zrows, zrows)],
                        acc.at[pl.ds(s * zrows, zrows)])
        plsc.subcore_barrier()

        def dst_sl(i):
            return dst_ref.at[pl.ds((c * _NSUB + s) * ept + i * _KEDGE,
                                    _KEDGE)]

        def start_idx(i, b):
            pltpu.async_copy(dst_sl(i), dstv[b], si[b])

        def wait_idx(i, b):
            pltpu.make_async_copy(dst_sl(i), dstv[b], si[b]).wait()

        def start_scatter(b):
            pltpu.async_copy(ones, acc.at[dstv[b]], ss[b], add=True)

        def wait_scatter(b):
            pltpu.make_async_copy(ones, acc.at[dstv[b]], ss[b]).wait()

        def step(i, b, static):
            wait_idx(i, b)
            start_scatter(b)

            def do_ws():
                wait_scatter(1 - b)

            def do_idx():
                start_idx(i + 1, 1 - b)

            if static:
                if i > 0:
                    do_ws()
                if i + 1 < nblk:
                    do_idx()
            else:
                pl.when(i > 0)(do_ws)
                pl.when(i + 1 < nblk)(do_idx)

        start_idx(0, 0)
        nsup, rem = nblk // 2, nblk % 2

        def sup_body(sup, carry):
            for j in range(2):
                step(sup * 2 + j, j, False)
            return carry

        lax.fori_loop(0, nsup, sup_body, 0)
        for j in range(rem):
            step(nsup * 2 + j, j, True)
        wait_scatter((nblk - 1) % 2)
        plsc.subcore_barrier()
        pltpu.sync_copy(acc.at[pl.ds(s * zrows, zrows)],
                        out_ref.at[pl.ds(c * acc_rows + s * zrows, zrows)])

    return pl.kernel(
        body,
        out_type=jax.ShapeDtypeStruct((_NSC * acc_rows, cw), jnp.float32),
        mesh=plsc.VectorSubcoreMesh(core_axis_name="c", subcore_axis_name="s"),
        scratch_types=[
            pltpu.VMEM((_KEDGE,), jnp.int32),
            pltpu.VMEM((_KEDGE,), jnp.int32),
            pltpu.VMEM((_KEDGE, cw), jnp.float32),
            pltpu.VMEM_SHARED((acc_rows, cw), jnp.float32),
            pltpu.SemaphoreType.DMA,
            pltpu.SemaphoreType.DMA,
            pltpu.SemaphoreType.DMA,
            pltpu.SemaphoreType.DMA,
        ],
        compiler_params=pltpu.CompilerParams(use_tc_tiling_on_sc=False),
    )


_BM = 1000


def _proj_int_body(x_ref, w_ref, b_ref, out_ref, outq_ref):
    y = jnp.dot(x_ref[...], w_ref[...],
                preferred_element_type=jnp.float32) + b_ref[...]
    out_ref[...] = y
    for q in range(4):
        outq_ref[q] = y[:, 32 * q:32 * (q + 1)]


def _proj_int(x, w, b):
    nb = x.shape[0] // _BM
    return pl.pallas_call(
        _proj_int_body,
        grid=(nb,),
        in_specs=[
            pl.BlockSpec((_BM, 128), lambda i: (i, 0)),
            pl.BlockSpec((128, 128), lambda i: (0, 0)),
            pl.BlockSpec((1, 128), lambda i: (0, 0)),
        ],
        out_specs=[
            pl.BlockSpec((_BM, 128), lambda i: (i, 0)),
            pl.BlockSpec((4, _BM, 32), lambda i: (0, i, 0)),
        ],
        out_shape=[
            jax.ShapeDtypeStruct((x.shape[0], 128), jnp.float32),
            jax.ShapeDtypeStruct((4, x.shape[0], 32), jnp.float32),
        ],
    )(x, w, b)


def _csum(cnt_blk):
    return cnt_blk[0, :, 0:1] + cnt_blk[1, :, 0:1]


def _const_build_body(al_ref, as_ref, ai_ref, cl_ref, cs_ref, ci_ref,
                      wls_ref, winj_ref, bl_ref, bs_ref, bi_ref, wfwi_ref,
                      out_ref):
    cl = _csum(cl_ref[...])
    cs = _csum(cs_ref[...])
    ci = _csum(ci_ref[...])
    rl = 1.0 / jnp.maximum(cl, 1.0)
    rs = 1.0 / jnp.maximum(cs, 1.0)
    ri = 1.0 / jnp.maximum(ci, 1.0)
    albk = al_ref[...]
    asbk = as_ref[...]
    ml = jnp.concatenate([albk[q] for q in range(4)], axis=1) * rl
    ms = jnp.concatenate([asbk[q] for q in range(2)], axis=1) * rs
    mi = ai_ref[0] * ri
    u = jnp.dot(jnp.concatenate([ml, ms], axis=1), wls_ref[...],
                preferred_element_type=jnp.float32)
    u = u + jnp.where(cl > 0, 1.0, 0.0) * bl_ref[...]
    u = u + jnp.where(cs > 0, 1.0, 0.0) * bs_ref[...]
    v = jnp.dot(mi, winj_ref[...], preferred_element_type=jnp.float32)
    v = v + jnp.where(ci > 0, 1.0, 0.0) * bi_ref[...]
    out_ref[...] = jnp.dot(jnp.concatenate([u, v], axis=1), wfwi_ref[...],
                           preferred_element_type=jnp.float32)


def _const_build(agg_lane, agg_sens, agg_inj, cnt_fl, cnt_fs, cnt_inc,
                 w_ls, wp_inj, b_lane, b_sens, b_inj, wfwi):
    nb = _N // _BM
    return pl.pallas_call(
        _const_build_body,
        grid=(nb,),
        in_specs=[
            pl.BlockSpec((4, _BM, 16), lambda i: (0, i, 0)),
            pl.BlockSpec((2, _BM, 16), lambda i: (0, i, 0)),
            pl.BlockSpec((1, _BM, 16), lambda i: (0, i, 0)),
            pl.BlockSpec((2, _BM, 16), lambda i: (0, i, 0)),
            pl.BlockSpec((2, _BM, 16), lambda i: (0, i, 0)),
            pl.BlockSpec((2, _BM, 16), lambda i: (0, i, 0)),
            pl.BlockSpec((96, 128), lambda i: (0, 0)),
            pl.BlockSpec((16, 128), lambda i: (0, 0)),
            pl.BlockSpec((1, 128), lambda i: (0, 0)),
            pl.BlockSpec((1, 128), lambda i: (0, 0)),
            pl.BlockSpec((1, 128), lambda i: (0, 0)),
            pl.BlockSpec((256, 384), lambda i: (0, 0)),
        ],
        out_specs=pl.BlockSpec((_BM, 384), lambda i: (i, 0)),
        out_shape=jax.ShapeDtypeStruct((_N, 384), jnp.float32),
    )(agg_lane, agg_sens, agg_inj, cnt_fl, cnt_fs, cnt_inc,
      w_ls, wp_inj, b_lane, b_sens, b_inj, wfwi)


def _combine_body(h_ref, sp_ref, csp_ref, const_ref, ws_ref, wsp_ref, b_ref,
                  out_ref, outq_ref):
    csp = _csum(csp_ref[...])
    rsp = 1.0 / jnp.maximum(csp, 1.0)
    spbk = sp_ref[...]
    msp = jnp.concatenate([spbk[q] for q in range(4)], axis=1) * rsp
    y = jnp.dot(h_ref[...], ws_ref[...], preferred_element_type=jnp.float32)
    y = y + jnp.dot(msp, wsp_ref[...], preferred_element_type=jnp.float32)
    y = y + b_ref[...] + const_ref[...]
    y = jnp.where(y > 0, y, jnp.exp(jnp.minimum(y, 0.0)) - 1.0)
    out_ref[...] = y
    for q in range(4):
        outq_ref[q] = y[:, 32 * q:32 * (q + 1)]


def _combine(h, agg_sp, cnt_sp, const_all, w_self_l, w_sp_l, b_l, lidx):
    nb = _N // _BM
    return pl.pallas_call(
        _combine_body,
        grid=(nb,),
        in_specs=[
            pl.BlockSpec((_BM, 128), lambda i: (i, 0)),
            pl.BlockSpec((4, _BM, 32), lambda i: (0, i, 0)),
            pl.BlockSpec((2, _BM, 16), lambda i: (0, i, 0)),
            pl.BlockSpec((_BM, 128), lambda i, _l=lidx: (i, _l)),
            pl.BlockSpec((128, 128), lambda i: (0, 0)),
            pl.BlockSpec((128, 128), lambda i: (0, 0)),
            pl.BlockSpec((1, 128), lambda i: (0, 0)),
        ],
        out_specs=[
            pl.BlockSpec((_BM, 128), lambda i: (i, 0)),
            pl.BlockSpec((4, _BM, 32), lambda i: (0, i, 0)),
        ],
        out_shape=[
            jax.ShapeDtypeStruct((_N, 128), jnp.float32),
            jax.ShapeDtypeStruct((4, _N, 32), jnp.float32),
        ],
    )(h, agg_sp, cnt_sp, const_all, w_self_l, w_sp_l, b_l)


def _prep_edges(edges, n_src, nch):
    src, dst = edges[0], edges[1]
    e = src.shape[0]
    e_pad = _pad_edges(e)
    pad = e_pad - e
    src_p = jnp.concatenate([src, jnp.zeros((pad,), jnp.int32)])
    dst_p = jnp.concatenate(
        [dst, _N + (jnp.arange(pad, dtype=jnp.int32) % _DUMMY)])
    srcq = (src_p[None, :]
            + (jnp.arange(nch, dtype=jnp.int32) * n_src)[:, None]).reshape(-1)
    return srcq, dst_p, e_pad


def _chunk16(x, nch):
    n = x.shape[0]
    return jnp.transpose(x.reshape(n, nch, 16), (1, 0, 2)).reshape(nch * n, 16)


def kernel(x_int, x_lane, x_sens, x_inj, Wp_int, bp_int, Wp_lane, bp_lane,
           Wp_sens, bp_sens, Wp_inj, bp_inj, W_self, b_self, W_spatial,
           W_flow, W_incident, spatial_e, flow_lane_e, flow_sens_e,
           incident_e):
    f32 = jnp.float32
    zeros16 = jnp.zeros((_N + _DUMMY, 16), f32)
    zeros32 = jnp.zeros((_N + _DUMMY, 32), f32)

    sp_srcq, sp_dst, sp_ep = _prep_edges(spatial_e, _N, 4)
    fl_srcq, fl_dst, fl_ep = _prep_edges(flow_lane_e, 50000, 4)
    fs_srcq, fs_dst, fs_ep = _prep_edges(flow_sens_e, 20000, 2)
    inc_srcq, inc_dst, inc_ep = _prep_edges(incident_e, 10000, 1)

    cnt_sp = _make_counts(_N, sp_ep)(sp_dst, zeros16)
    cnt_fl = _make_counts(_N, fl_ep)(fl_dst, zeros16)
    cnt_fs = _make_counts(_N, fs_ep)(fs_dst, zeros16)
    cnt_inc = _make_counts(_N, inc_ep)(inc_dst, zeros16)

    agg_lane = _make_seg_sum(50000, _N, fl_ep, 16, 4)(
        _chunk16(x_lane, 4), fl_srcq, fl_dst, zeros16)
    agg_sens = _make_seg_sum(20000, _N, fs_ep, 16, 2)(
        _chunk16(x_sens, 2), fs_srcq, fs_dst, zeros16)
    agg_inj = _make_seg_sum(10000, _N, inc_ep, 16, 1)(
        _chunk16(x_inj, 1), inc_srcq, inc_dst, zeros16)

    h, hq = _proj_int(x_int, Wp_int, bp_int.reshape(1, 128))

    w_ls = jnp.concatenate([Wp_lane, Wp_sens], axis=0)
    wf3 = jnp.transpose(W_flow, (1, 0, 2)).reshape(128, 384)
    wi3 = jnp.transpose(W_incident, (1, 0, 2)).reshape(128, 384)
    wfwi = jnp.concatenate([wf3, wi3], axis=0)
    const_all = _const_build(
        agg_lane.reshape(4, _NP, 16), agg_sens.reshape(2, _NP, 16),
        agg_inj.reshape(1, _NP, 16),
        cnt_fl.reshape(2, _NP, 16), cnt_fs.reshape(2, _NP, 16),
        cnt_inc.reshape(2, _NP, 16),
        w_ls, Wp_inj, bp_lane.reshape(1, 128), bp_sens.reshape(1, 128),
        bp_inj.reshape(1, 128), wfwi)

    cnt_sp_r = cnt_sp.reshape(2, _NP, 16)
    seg_sp = _make_seg_sum(_N, _N, sp_ep, 32, 4)
    for l in range(_L):
        agg_sp = seg_sp(hq.reshape(4 * _N, 32), sp_srcq, sp_dst, zeros32)
        h, hq = _combine(h, agg_sp.reshape(4, _NP, 32), cnt_sp_r, const_all,
                         W_self[l], W_spatial[l], b_self[l].reshape(1, 128),
                         l)
    return h
